# baseline (speedup 1.0000x reference)
import math

import jax
import jax.numpy as jnp
from jax import lax
from jax.experimental import pallas as pl
from jax.experimental.pallas import tpu as pltpu
from jax.experimental.pallas import tpu_sc as plsc

VOCAB = 100000
D = 128
SCALE = math.sqrt(float(D))

NC = 2
NS = 16
NW = NC * NS
BATCH = 4096
SEQ = 50
B_PER_W = BATCH // NW
NB = 5
LA = 2


def _sc_embed(table, idx3):
    mesh = plsc.VectorSubcoreMesh(core_axis_name="c", subcore_axis_name="s")

    def body(table_hbm, idx_hbm, out_hbm, idx_v, bufs, *sems):
        gsem = sems[:NB]
        ssem = sems[NB:]
        wid = lax.axis_index("s") * NC + lax.axis_index("c")
        base = wid * B_PER_W
        pltpu.sync_copy(idx_hbm.at[wid], idx_v)

        def gather(c, b):
            pltpu.async_copy(
                table_hbm.at[idx_v.at[c]], bufs.at[b], gsem[b])

        def gather_wait(c, b):
            pltpu.make_async_copy(
                table_hbm.at[idx_v.at[c]], bufs.at[b], gsem[b]).wait()

        def scatter(c, b):
            pltpu.async_copy(
                bufs.at[b], out_hbm.at[c, pl.ds(base, B_PER_W)], ssem[b])

        def scatter_wait(c, b):
            pltpu.make_async_copy(
                bufs.at[b], out_hbm.at[c, pl.ds(base, B_PER_W)],
                ssem[b]).wait()

        def compute(c, b):
            gather_wait(c, b)

            def row_body(r, carry):
                for j in range(D // 16):
                    sl = pl.ds(j * 16, 16)
                    bufs[b, r, sl] = bufs[b, r, sl] * SCALE
                return carry

            lax.fori_loop(0, B_PER_W, row_body, 0, unroll=4)
            scatter(c, b)

        for b in range(LA):
            gather(b, b)
        for db in range(NB):
            c2 = db + LA
            if c2 >= NB:
                scatter_wait(c2 - NB, c2 % NB)
            gather(c2, c2 % NB)
            compute(db, db)

        def group_body(i, carry):
            g = i * NB
            for db in range(NB):
                c = g + db
                b2 = (db + LA) % NB
                scatter_wait(c + LA - NB, b2)
                gather(c + LA, b2)
                compute(c, db)
            return carry

        lax.fori_loop(1, SEQ // NB - 1, group_body, 0, unroll=False)

        g = SEQ - NB
        for db in range(NB):
            c = g + db
            c2 = c + LA
            if c2 < SEQ:
                scatter_wait(c2 - NB, c2 % NB)
                gather(c2, c2 % NB)
            compute(c, db)
        for db in range(NB):
            scatter_wait(g + db, db)

    run = pl.kernel(
        body,
        out_type=jax.ShapeDtypeStruct((SEQ, BATCH, D), jnp.float32),
        mesh=mesh,
        scratch_types=(
            [pltpu.VMEM((SEQ, B_PER_W), jnp.int32),
             pltpu.VMEM((NB, B_PER_W, D), jnp.float32)]
            + [pltpu.SemaphoreType.DMA] * (2 * NB)
        ),
        compiler_params=pltpu.CompilerParams(use_tc_tiling_on_sc=True),
    )
    return run(table, idx3)


def kernel(x, weight):
    idx3 = x.T.astype(jnp.int32).reshape(SEQ, NW, B_PER_W).transpose(1, 0, 2)
    out = _sc_embed(weight, idx3)
    return out.transpose(1, 0, 2)


# speedup vs baseline: 1.0265x; 1.0265x over previous
import math

import jax
import jax.numpy as jnp
from jax import lax
from jax.experimental import pallas as pl
from jax.experimental.pallas import tpu as pltpu
from jax.experimental.pallas import tpu_sc as plsc

VOCAB = 100000
D = 128
SCALE = math.sqrt(float(D))

NC = 2
NS = 16
NW = NC * NS
BATCH = 4096
SEQ = 50
B_PER_W = BATCH // NW
NB = 5
LA = 3


def _sc_embed(table, idx3):
    mesh = plsc.VectorSubcoreMesh(core_axis_name="c", subcore_axis_name="s")

    def body(table_hbm, idx_hbm, out_hbm, idx_v, bufs, *sems):
        gsem = sems[:NB]
        ssem = sems[NB:]
        wid = lax.axis_index("s") * NC + lax.axis_index("c")
        base = wid * B_PER_W
        pltpu.sync_copy(idx_hbm.at[wid], idx_v)

        def gather(c, b):
            pltpu.async_copy(
                table_hbm.at[idx_v.at[c]], bufs.at[b], gsem[b])

        def gather_wait(c, b):
            pltpu.make_async_copy(
                table_hbm.at[idx_v.at[c]], bufs.at[b], gsem[b]).wait()

        def scatter(c, b):
            pltpu.async_copy(
                bufs.at[b], out_hbm.at[c, pl.ds(base, B_PER_W)], ssem[b])

        def scatter_wait(c, b):
            pltpu.make_async_copy(
                bufs.at[b], out_hbm.at[c, pl.ds(base, B_PER_W)],
                ssem[b]).wait()

        def compute(c, b):
            gather_wait(c, b)

            def row_body(r, carry):
                for j in range(D // 16):
                    sl = pl.ds(j * 16, 16)
                    bufs[b, r, sl] = bufs[b, r, sl] * SCALE
                return carry

            lax.fori_loop(0, B_PER_W, row_body, 0, unroll=False)
            scatter(c, b)

        for b in range(LA):
            gather(b, b)
        for db in range(NB):
            c2 = db + LA
            if c2 >= NB:
                scatter_wait(c2 - NB, c2 % NB)
            gather(c2, c2 % NB)
            compute(db, db)

        def group_body(i, carry):
            g = i * NB
            for db in range(NB):
                c = g + db
                b2 = (db + LA) % NB
                scatter_wait(c + LA - NB, b2)
                gather(c + LA, b2)
                compute(c, db)
            return carry

        lax.fori_loop(1, SEQ // NB - 1, group_body, 0, unroll=False)

        g = SEQ - NB
        for db in range(NB):
            c = g + db
            c2 = c + LA
            if c2 < SEQ:
                scatter_wait(c2 - NB, c2 % NB)
                gather(c2, c2 % NB)
            compute(c, db)
        for db in range(NB):
            scatter_wait(g + db, db)

    run = pl.kernel(
        body,
        out_type=jax.ShapeDtypeStruct((SEQ, BATCH, D), jnp.float32),
        mesh=mesh,
        scratch_types=(
            [pltpu.VMEM((SEQ, B_PER_W), jnp.int32),
             pltpu.VMEM((NB, B_PER_W, D), jnp.float32)]
            + [pltpu.SemaphoreType.DMA] * (2 * NB)
        ),
        compiler_params=pltpu.CompilerParams(use_tc_tiling_on_sc=True),
    )
    return run(table, idx3)


def kernel(x, weight):
    idx3 = x.T.astype(jnp.int32).reshape(SEQ, NW, B_PER_W).transpose(1, 0, 2)
    out = _sc_embed(weight, idx3)
    return out.transpose(1, 0, 2)


# speedup vs baseline: 1.0265x; 1.0000x over previous
import math

import jax
import jax.numpy as jnp
from jax import lax
from jax.experimental import pallas as pl
from jax.experimental.pallas import tpu as pltpu
from jax.experimental.pallas import tpu_sc as plsc

VOCAB = 100000
D = 128
SCALE = math.sqrt(float(D))

NC = 2
NS = 16
NW = NC * NS
BATCH = 4096
SEQ = 50
B_PER_W = BATCH // NW
NB = 5
LA = 4


def _sc_embed(table, idx3):
    mesh = plsc.VectorSubcoreMesh(core_axis_name="c", subcore_axis_name="s")

    def body(table_hbm, idx_hbm, out_hbm, idx_v, bufs, *sems):
        gsem = sems[:NB]
        ssem = sems[NB:]
        wid = lax.axis_index("s") * NC + lax.axis_index("c")
        base = wid * B_PER_W
        pltpu.sync_copy(idx_hbm.at[wid], idx_v)

        def gather(c, b):
            pltpu.async_copy(
                table_hbm.at[idx_v.at[c]], bufs.at[b], gsem[b])

        def gather_wait(c, b):
            pltpu.make_async_copy(
                table_hbm.at[idx_v.at[c]], bufs.at[b], gsem[b]).wait()

        def scatter(c, b):
            pltpu.async_copy(
                bufs.at[b], out_hbm.at[c, pl.ds(base, B_PER_W)], ssem[b])

        def scatter_wait(c, b):
            pltpu.make_async_copy(
                bufs.at[b], out_hbm.at[c, pl.ds(base, B_PER_W)],
                ssem[b]).wait()

        def compute(c, b):
            gather_wait(c, b)

            def row_body(r, carry):
                for j in range(D // 16):
                    sl = pl.ds(j * 16, 16)
                    bufs[b, r, sl] = bufs[b, r, sl] * SCALE
                return carry

            lax.fori_loop(0, B_PER_W, row_body, 0, unroll=False)
            scatter(c, b)

        for b in range(LA):
            gather(b, b)
        for db in range(NB):
            c2 = db + LA
            if c2 >= NB:
                scatter_wait(c2 - NB, c2 % NB)
            gather(c2, c2 % NB)
            compute(db, db)

        def group_body(i, carry):
            g = i * NB
            for db in range(NB):
                c = g + db
                b2 = (db + LA) % NB
                scatter_wait(c + LA - NB, b2)
                gather(c + LA, b2)
                compute(c, db)
            return carry

        lax.fori_loop(1, SEQ // NB - 1, group_body, 0, unroll=False)

        g = SEQ - NB
        for db in range(NB):
            c = g + db
            c2 = c + LA
            if c2 < SEQ:
                scatter_wait(c2 - NB, c2 % NB)
                gather(c2, c2 % NB)
            compute(c, db)
        for db in range(NB):
            scatter_wait(g + db, db)

    run = pl.kernel(
        body,
        out_type=jax.ShapeDtypeStruct((SEQ, BATCH, D), jnp.float32),
        mesh=mesh,
        scratch_types=(
            [pltpu.VMEM((SEQ, B_PER_W), jnp.int32),
             pltpu.VMEM((NB, B_PER_W, D), jnp.float32)]
            + [pltpu.SemaphoreType.DMA] * (2 * NB)
        ),
        compiler_params=pltpu.CompilerParams(use_tc_tiling_on_sc=True),
    )
    return run(table, idx3)


def kernel(x, weight):
    idx3 = x.T.astype(jnp.int32).reshape(SEQ, NW, B_PER_W).transpose(1, 0, 2)
    out = _sc_embed(weight, idx3)
    return out.transpose(1, 0, 2)


# speedup vs baseline: 1.0341x; 1.0074x over previous
import math

import jax
import jax.numpy as jnp
from jax import lax
from jax.experimental import pallas as pl
from jax.experimental.pallas import tpu as pltpu
from jax.experimental.pallas import tpu_sc as plsc

VOCAB = 100000
D = 128
SCALE = math.sqrt(float(D))

NC = 2
NS = 16
NW = NC * NS
BATCH = 4096
SEQ = 50
B_PER_W = BATCH // NW
NB = 5
LA = 4


def _sc_embed(table, idx3):
    mesh = plsc.VectorSubcoreMesh(core_axis_name="c", subcore_axis_name="s")

    def body(table_hbm, idx_hbm, out_hbm, idx_v, bufs, *sems):
        gsem = sems[:NB]
        gsem2 = sems[NB:2 * NB]
        ssem = sems[2 * NB:]
        wid = lax.axis_index("s") * NC + lax.axis_index("c")
        base = wid * B_PER_W
        pltpu.sync_copy(idx_hbm.at[wid], idx_v)

        H = B_PER_W // 2

        def gather(c, b):
            pltpu.async_copy(
                table_hbm.at[idx_v.at[c].at[pl.ds(0, H)]],
                bufs.at[b].at[pl.ds(0, H)], gsem[b])
            pltpu.async_copy(
                table_hbm.at[idx_v.at[c].at[pl.ds(H, H)]],
                bufs.at[b].at[pl.ds(H, H)], gsem2[b])

        def gather_wait(c, b):
            pltpu.make_async_copy(
                table_hbm.at[idx_v.at[c].at[pl.ds(0, H)]],
                bufs.at[b].at[pl.ds(0, H)], gsem[b]).wait()
            pltpu.make_async_copy(
                table_hbm.at[idx_v.at[c].at[pl.ds(H, H)]],
                bufs.at[b].at[pl.ds(H, H)], gsem2[b]).wait()

        def scatter(c, b):
            pltpu.async_copy(
                bufs.at[b], out_hbm.at[c, pl.ds(base, B_PER_W)], ssem[b])

        def scatter_wait(c, b):
            pltpu.make_async_copy(
                bufs.at[b], out_hbm.at[c, pl.ds(base, B_PER_W)],
                ssem[b]).wait()

        def compute(c, b):
            gather_wait(c, b)

            def row_body(r, carry):
                for j in range(D // 16):
                    sl = pl.ds(j * 16, 16)
                    bufs[b, r, sl] = bufs[b, r, sl] * SCALE
                return carry

            lax.fori_loop(0, B_PER_W, row_body, 0, unroll=False)
            scatter(c, b)

        for b in range(LA):
            gather(b, b)
        for db in range(NB):
            c2 = db + LA
            if c2 >= NB:
                scatter_wait(c2 - NB, c2 % NB)
            gather(c2, c2 % NB)
            compute(db, db)

        def group_body(i, carry):
            g = i * NB
            for db in range(NB):
                c = g + db
                b2 = (db + LA) % NB
                scatter_wait(c + LA - NB, b2)
                gather(c + LA, b2)
                compute(c, db)
            return carry

        lax.fori_loop(1, SEQ // NB - 1, group_body, 0, unroll=False)

        g = SEQ - NB
        for db in range(NB):
            c = g + db
            c2 = c + LA
            if c2 < SEQ:
                scatter_wait(c2 - NB, c2 % NB)
                gather(c2, c2 % NB)
            compute(c, db)
        for db in range(NB):
            scatter_wait(g + db, db)

    run = pl.kernel(
        body,
        out_type=jax.ShapeDtypeStruct((SEQ, BATCH, D), jnp.float32),
        mesh=mesh,
        scratch_types=(
            [pltpu.VMEM((SEQ, B_PER_W), jnp.int32),
             pltpu.VMEM((NB, B_PER_W, D), jnp.float32)]
            + [pltpu.SemaphoreType.DMA] * (3 * NB)
        ),
        compiler_params=pltpu.CompilerParams(use_tc_tiling_on_sc=True),
    )
    return run(table, idx3)


def kernel(x, weight):
    idx3 = x.T.astype(jnp.int32).reshape(SEQ, NW, B_PER_W).transpose(1, 0, 2)
    out = _sc_embed(weight, idx3)
    return out.transpose(1, 0, 2)


# speedup vs baseline: 1.0347x; 1.0006x over previous
import math

import jax
import jax.numpy as jnp
from jax import lax
from jax.experimental import pallas as pl
from jax.experimental.pallas import tpu as pltpu
from jax.experimental.pallas import tpu_sc as plsc

VOCAB = 100000
D = 128
SCALE = math.sqrt(float(D))

NC = 2
NS = 16
NW = NC * NS
BATCH = 4096
SEQ = 50
B_PER_W = BATCH // NW
NB = 5
LA = 4


def _sc_embed(table, idx3):
    mesh = plsc.VectorSubcoreMesh(core_axis_name="c", subcore_axis_name="s")

    def body(table_hbm, idx_hbm, out_hbm, idx_v, bufs, *sems):
        gsem = sems[:NB]
        gsem2 = sems[NB:2 * NB]
        ssem = sems[2 * NB:]
        wid = lax.axis_index("s") * NC + lax.axis_index("c")
        base = wid * B_PER_W
        pltpu.sync_copy(idx_hbm.at[wid], idx_v)

        H = B_PER_W // 4

        def gather(c, b):
            for k in range(4):
                pltpu.async_copy(
                    table_hbm.at[idx_v.at[c].at[pl.ds(k * H, H)]],
                    bufs.at[b].at[pl.ds(k * H, H)],
                    (gsem if k % 2 == 0 else gsem2)[b])

        def gather_wait(c, b):
            for k in range(4):
                pltpu.make_async_copy(
                    table_hbm.at[idx_v.at[c].at[pl.ds(k * H, H)]],
                    bufs.at[b].at[pl.ds(k * H, H)],
                    (gsem if k % 2 == 0 else gsem2)[b]).wait()

        def scatter(c, b):
            pltpu.async_copy(
                bufs.at[b], out_hbm.at[c, pl.ds(base, B_PER_W)], ssem[b])

        def scatter_wait(c, b):
            pltpu.make_async_copy(
                bufs.at[b], out_hbm.at[c, pl.ds(base, B_PER_W)],
                ssem[b]).wait()

        def compute(c, b):
            gather_wait(c, b)

            def row_body(r, carry):
                for j in range(D // 16):
                    sl = pl.ds(j * 16, 16)
                    bufs[b, r, sl] = bufs[b, r, sl] * SCALE
                return carry

            lax.fori_loop(0, B_PER_W, row_body, 0, unroll=False)
            scatter(c, b)

        for b in range(LA):
            gather(b, b)
        for db in range(NB):
            c2 = db + LA
            if c2 >= NB:
                scatter_wait(c2 - NB, c2 % NB)
            gather(c2, c2 % NB)
            compute(db, db)

        def group_body(i, carry):
            g = i * NB
            for db in range(NB):
                c = g + db
                b2 = (db + LA) % NB
                scatter_wait(c + LA - NB, b2)
                gather(c + LA, b2)
                compute(c, db)
            return carry

        lax.fori_loop(1, SEQ // NB - 1, group_body, 0, unroll=False)

        g = SEQ - NB
        for db in range(NB):
            c = g + db
            c2 = c + LA
            if c2 < SEQ:
                scatter_wait(c2 - NB, c2 % NB)
                gather(c2, c2 % NB)
            compute(c, db)
        for db in range(NB):
            scatter_wait(g + db, db)

    run = pl.kernel(
        body,
        out_type=jax.ShapeDtypeStruct((SEQ, BATCH, D), jnp.float32),
        mesh=mesh,
        scratch_types=(
            [pltpu.VMEM((SEQ, B_PER_W), jnp.int32),
             pltpu.VMEM((NB, B_PER_W, D), jnp.float32)]
            + [pltpu.SemaphoreType.DMA] * (3 * NB)
        ),
        compiler_params=pltpu.CompilerParams(use_tc_tiling_on_sc=True),
    )
    return run(table, idx3)


def kernel(x, weight):
    idx3 = x.T.astype(jnp.int32).reshape(SEQ, NW, B_PER_W).transpose(1, 0, 2)
    out = _sc_embed(weight, idx3)
    return out.transpose(1, 0, 2)
